# Initial kernel scaffold; baseline (speedup 1.0000x reference)
import jax
import jax.numpy as jnp
from jax.experimental import pallas as pl


def kernel(x, seg, params):
    raise NotImplementedError("write your pallas kernel here")


# speedup vs baseline: 5.9439x; 5.9439x over previous
import jax
import jax.numpy as jnp
from jax.experimental import pallas as pl

_N = 32768
_D = 256
_D_OUT = 128
_NSEG = 16
_BLK = 1024
_NBLK = _N // _BLK
_EPS = 1e-5


def _oh_pair(seg_blk):
    i1 = jax.lax.broadcasted_iota(jnp.int32, (seg_blk.shape[0], _NSEG), 1)
    oh = (seg_blk[:, None] == i1).astype(jnp.float32)
    i0 = jax.lax.broadcasted_iota(jnp.int32, (_NSEG, seg_blk.shape[0]), 0)
    oht = (seg_blk[None, :] == i0).astype(jnp.float32)
    return oh, oht


def _dot(a, b):
    return jnp.dot(a, b, preferred_element_type=jnp.float32)


def _k_first(seg_ref, x_ref, w1_ref, b1_ref, w2_ref, b2_ref,
             xphi_ref, ssh_ref, ssx_ref, sumsq_ref, cnt_ref):
    seg_blk = seg_ref[0, 0, :]
    oh, oht = _oh_pair(seg_blk)
    x = x_ref[...]
    t = jnp.maximum(_dot(x, w1_ref[...]) + b1_ref[...], 0.0)
    xp = _dot(t, w2_ref[...]) + b2_ref[...]
    xphi_ref[...] = xp

    @pl.when(pl.program_id(0) == 0)
    def _():
        ssh_ref[...] = jnp.zeros_like(ssh_ref)
        ssx_ref[...] = jnp.zeros_like(ssx_ref)
        sumsq_ref[...] = jnp.zeros_like(sumsq_ref)
        cnt_ref[...] = jnp.zeros_like(cnt_ref)

    ssh_ref[...] += _dot(oht, x)
    ssx_ref[...] += _dot(oht, xp)
    sumsq_ref[...] += jnp.sum(xp * xp, axis=0, keepdims=True)
    cnt_ref[...] += jnp.sum(oh, axis=0, keepdims=True)


def _k_fused(seg_ref, h_ref, xp_ref, a_ref, c_ref,
             w1_ref, b1_ref, w2_ref, b2_ref,
             hn_ref, xpn_ref, ssx_ref, sumsq_ref):
    seg_blk = seg_ref[0, 0, :]
    oh, oht = _oh_pair(seg_blk)
    hn = h_ref[...] + xp_ref[...] * a_ref[...] + _dot(oh, c_ref[...])
    hn_ref[...] = hn
    t = jnp.maximum(_dot(hn, w1_ref[...]) + b1_ref[...], 0.0)
    xpn = _dot(t, w2_ref[...]) + b2_ref[...]
    xpn_ref[...] = xpn

    @pl.when(pl.program_id(0) == 0)
    def _():
        ssx_ref[...] = jnp.zeros_like(ssx_ref)
        sumsq_ref[...] = jnp.zeros_like(sumsq_ref)

    ssx_ref[...] += _dot(oht, xpn)
    sumsq_ref[...] += jnp.sum(xpn * xpn, axis=0, keepdims=True)


def _k_last(seg_ref, h_ref, xp_ref, a_ref, c_ref,
            w1_ref, b1_ref, w2_ref, b2_ref, sp_ref):
    seg_blk = seg_ref[0, 0, :]
    oh, oht = _oh_pair(seg_blk)
    hn = h_ref[...] + xp_ref[...] * a_ref[...] + _dot(oh, c_ref[...])
    t = jnp.maximum(_dot(hn, w1_ref[...]) + b1_ref[...], 0.0)
    xpp = _dot(t, w2_ref[...]) + b2_ref[...]

    @pl.when(pl.program_id(0) == 0)
    def _():
        sp_ref[...] = jnp.zeros_like(sp_ref)

    sp_ref[...] += _dot(oht, xpp)


def _k_tiny(ssh_ref, ssx_ref, sumsq_ref, cnt_ref,
            w1_ref, b1_ref, w2_ref, b2_ref, g_ref, bb_ref,
            a_ref, c_ref, sshn_ref):
    s = ssh_ref[...]
    t = jnp.maximum(_dot(s, w1_ref[...]) + b1_ref[...], 0.0)
    r = _dot(t, w2_ref[...]) + b2_ref[...]
    ssx = ssx_ref[...]
    cnt_t = cnt_ref[...].reshape(_NSEG, 1)
    sum_xphi = jnp.sum(ssx, axis=0, keepdims=True)
    mean = (sum_xphi + jnp.sum(cnt_t * r, axis=0, keepdims=True)) / _N
    ez2 = (sumsq_ref[...]
           + 2.0 * jnp.sum(r * ssx, axis=0, keepdims=True)
           + jnp.sum(cnt_t * (r * r), axis=0, keepdims=True)) / _N
    var = ez2 - mean * mean
    a = g_ref[...] / jnp.sqrt(var + _EPS)
    bshift = bb_ref[...] - mean * a
    c = r * a + bshift
    a_ref[...] = a
    c_ref[...] = c
    sshn_ref[...] = s + ssx * a + cnt_t * c


def _k_out(sp_ref, w1_ref, b1_ref, w2_ref, b2_ref, out_ref):
    t = jnp.maximum(_dot(sp_ref[...], w1_ref[...]) + b1_ref[...], 0.0)
    out_ref[...] = _dot(t, w2_ref[...]) + b2_ref[...]


def _f32(shape):
    return jax.ShapeDtypeStruct(shape, jnp.float32)


def _mw(p):
    return p["W1"], p["b1"].reshape(1, -1), p["W2"], p["b2"].reshape(1, -1)


_SEG_SPEC = pl.BlockSpec((1, 1, _BLK), lambda i: (i, 0, 0))
_ROW_SPEC = pl.BlockSpec((_BLK, _D), lambda i: (i, 0))
_W_SPEC = pl.BlockSpec((_D, _D), lambda i: (0, 0))
_B_SPEC = pl.BlockSpec((1, _D), lambda i: (0, 0))
_A_SPEC = pl.BlockSpec((1, _D), lambda i: (0, 0))
_C_SPEC = pl.BlockSpec((_NSEG, _D), lambda i: (0, 0))
_SS_SPEC = pl.BlockSpec((_NSEG, _D), lambda i: (0, 0))
_COL_SPEC = pl.BlockSpec((1, _D), lambda i: (0, 0))
_CNT_SPEC = pl.BlockSpec((1, _NSEG), lambda i: (0, 0))


def kernel(x, seg, params):
    seg3 = seg.astype(jnp.int32).reshape(_NBLK, 1, _BLK)
    layers = params["layers"]
    pool = params["pooling"]

    first = pl.pallas_call(
        _k_first,
        grid=(_NBLK,),
        in_specs=[_SEG_SPEC, _ROW_SPEC, _W_SPEC, _B_SPEC, _W_SPEC, _B_SPEC],
        out_specs=[_ROW_SPEC, _SS_SPEC, _SS_SPEC, _COL_SPEC, _CNT_SPEC],
        out_shape=[_f32((_N, _D)), _f32((_NSEG, _D)), _f32((_NSEG, _D)),
                   _f32((1, _D)), _f32((1, _NSEG))],
    )
    fused = pl.pallas_call(
        _k_fused,
        grid=(_NBLK,),
        in_specs=[_SEG_SPEC, _ROW_SPEC, _ROW_SPEC, _A_SPEC, _C_SPEC,
                  _W_SPEC, _B_SPEC, _W_SPEC, _B_SPEC],
        out_specs=[_ROW_SPEC, _ROW_SPEC, _SS_SPEC, _COL_SPEC],
        out_shape=[_f32((_N, _D)), _f32((_N, _D)), _f32((_NSEG, _D)),
                   _f32((1, _D))],
    )
    last = pl.pallas_call(
        _k_last,
        grid=(_NBLK,),
        in_specs=[_SEG_SPEC, _ROW_SPEC, _ROW_SPEC, _A_SPEC, _C_SPEC,
                  _W_SPEC, _B_SPEC, _W_SPEC, _B_SPEC],
        out_specs=[_SS_SPEC],
        out_shape=[_f32((_NSEG, _D))],
    )
    tiny = pl.pallas_call(
        _k_tiny,
        out_shape=[_f32((1, _D)), _f32((_NSEG, _D)), _f32((_NSEG, _D))],
    )
    kout = pl.pallas_call(
        _k_out,
        out_shape=_f32((_NSEG, _D_OUT)),
    )

    def bn(L):
        return L["bn_g"].reshape(1, -1), L["bn_b"].reshape(1, -1)

    xphi, ssh, ssx, sumsq, cnt = first(seg3, x, *_mw(layers[0]["phi"]))
    a, c, ssh = tiny(ssh, ssx, sumsq, cnt, *_mw(layers[0]["rho"]), *bn(layers[0]))
    h, xphi, ssx, sumsq = fused(seg3, x, xphi, a, c, *_mw(layers[1]["phi"]))
    a, c, ssh = tiny(ssh, ssx, sumsq, cnt, *_mw(layers[1]["rho"]), *bn(layers[1]))
    h, xphi, ssx, sumsq = fused(seg3, h, xphi, a, c, *_mw(layers[2]["phi"]))
    a, c, _ = tiny(ssh, ssx, sumsq, cnt, *_mw(layers[2]["rho"]), *bn(layers[2]))
    (sp,) = last(seg3, h, xphi, a, c, *_mw(pool["phi"]))
    out = kout(sp, *_mw(pool["rho"]))
    return out
